# baseline (speedup 1.0000x reference)
import functools

import jax
import jax.numpy as jnp
from jax.experimental import pallas as pl
from jax.experimental.pallas import tpu as pltpu

_VMEM = pl.BlockSpec(memory_space=pltpu.MemorySpace.VMEM)


def _blk_body(x_ref, w1_ref, b1_ref, w2_ref, s2_ref, m_ref, o_ref,
              z_scr, p_scr, *, Wp, Nimg, Cin, Cmid):
    offs = tuple(dy * Wp + dx for dy in range(3) for dx in range(3))
    acc = None
    for t, off in enumerate(offs):
        xs = x_ref[:, off:off + Nimg]
        part = (w1_ref[t] * xs) if Cin == 1 else jnp.dot(
            w1_ref[t], xs, preferred_element_type=jnp.float32)
        acc = part if acc is None else acc + part
    acc = acc + b1_ref[...]

    z_scr[...] = jnp.zeros_like(z_scr)
    z_scr[:, Wp + 1:Wp + 1 + Nimg] = acc * m_ref[...]

    acc2 = None
    for t, off in enumerate(offs):
        p = jnp.dot(w2_ref[t], z_scr[:, off:off + Nimg],
                    preferred_element_type=jnp.float32)
        acc2 = p if acc2 is None else acc2 + p
    acc2 = jnp.maximum(acc2 + s2_ref[...], 0.0)

    p_scr[...] = jnp.zeros_like(p_scr)
    p_scr[:, 0:Nimg] = acc2
    o_ref[...] = jnp.maximum(
        jnp.maximum(p_scr[:, 0:Nimg], p_scr[:, 1:1 + Nimg]),
        jnp.maximum(p_scr[:, Wp:Wp + Nimg], p_scr[:, Wp + 1:Wp + 1 + Nimg]),
    ).astype(o_ref.dtype)


def _run_block(xf, w1t, b1, w2t, s2, mask, *, B, H, W, Cin, Cmid):
    Hp, Wp = H + 2, W + 2
    Nimg = Hp * Wp
    NP = Nimg + 2 * Wp + 2
    body = functools.partial(_blk_body, Wp=Wp, Nimg=Nimg, Cin=Cin, Cmid=Cmid)
    return pl.pallas_call(
        body,
        out_shape=jax.ShapeDtypeStruct((B, Cmid, Nimg), xf.dtype),
        grid=(B,),
        in_specs=[
            pl.BlockSpec((None, Cin, NP), lambda b: (b, 0, 0)),
            pl.BlockSpec((9, Cmid, Cin), lambda b: (0, 0, 0)),
            pl.BlockSpec((Cmid, 1), lambda b: (0, 0)),
            pl.BlockSpec((9, Cmid, Cmid), lambda b: (0, 0, 0)),
            pl.BlockSpec((Cmid, 1), lambda b: (0, 0)),
            pl.BlockSpec((1, Nimg), lambda b: (0, 0)),
        ],
        out_specs=pl.BlockSpec((None, Cmid, Nimg), lambda b: (b, 0, 0)),
        scratch_shapes=[
            pltpu.VMEM((Cmid, Nimg + 2 * Wp + 2), jnp.float32),
            pltpu.VMEM((Cmid, Nimg + Wp + 1), jnp.float32),
        ],
        compiler_params=pltpu.CompilerParams(
            dimension_semantics=("parallel",)),
    )(xf, w1t, b1, w2t, s2, mask)


def _fc_body(x_ref, w_ref, b_ref, o_ref):
    o_ref[...] = (jnp.dot(x_ref[...], w_ref[...],
                          preferred_element_type=jnp.float32)
                  + b_ref[...]).astype(o_ref.dtype)


def _pack(x):
    B, C, H, W = x.shape
    Wp = W + 2
    xp = jnp.pad(x, ((0, 0), (0, 0), (1, 1), (1, 1)))
    return jnp.pad(xp.reshape(B, C, (H + 2) * Wp),
                   ((0, 0), (0, 0), (0, 2 * Wp + 2)))


def _mask(H, W):
    Hp, Wp = H + 2, W + 2
    yy = jnp.arange(Hp)[:, None]
    xx = jnp.arange(Wp)[None, :]
    return ((yy < H) & (xx < W)).astype(jnp.float32).reshape(1, Hp * Wp)


def kernel(x, b0_w1t, b0_b1, b0_w2t, b0_shift2, b0_ref_w1, b0_ref_b1, b0_ref_w2, b0_ref_b2, b0_ref_gamma, b0_ref_beta, b0_ref_mean, b0_ref_var, b1_w1t, b1_b1, b1_w2t, b1_shift2, b1_ref_w1, b1_ref_b1, b1_ref_w2, b1_ref_b2, b1_ref_gamma, b1_ref_beta, b1_ref_mean, b1_ref_var, b2_w1t, b2_b1, b2_w2t, b2_shift2, b2_ref_w1, b2_ref_b1, b2_ref_w2, b2_ref_b2, b2_ref_gamma, b2_ref_beta, b2_ref_mean, b2_ref_var, b3_w1t, b3_b1, b3_w2t, b3_shift2, b3_ref_w1, b3_ref_b1, b3_ref_w2, b3_ref_b2, b3_ref_gamma, b3_ref_beta, b3_ref_mean, b3_ref_var, fc_wt, fc_b, ref_fc_w, ref_fc_b):
    B, _, H, W = x.shape
    blocks = (
        (b0_w1t, b0_b1, b0_w2t, b0_shift2),
        (b1_w1t, b1_b1, b1_w2t, b1_shift2),
        (b2_w1t, b2_b1, b2_w2t, b2_shift2),
        (b3_w1t, b3_b1, b3_w2t, b3_shift2),
    )
    cur = x
    for w1t, b1, w2t, s2 in blocks:
        Cmid, Cin = w1t.shape[1], w1t.shape[2]
        yf = _run_block(_pack(cur), w1t, b1, w2t, s2, _mask(H, W),
                        B=B, H=H, W=W, Cin=Cin, Cmid=Cmid)
        Hp, Wp = H + 2, W + 2
        cur = yf.reshape(B, Cmid, Hp, Wp)[:, :, 0:H:2, 0:W:2]
        H, W = H // 2, W // 2
    feats = cur.reshape(B, -1)
    return pl.pallas_call(
        _fc_body,
        out_shape=jax.ShapeDtypeStruct((B, fc_wt.shape[1]), feats.dtype),
        in_specs=[_VMEM] * 3,
        out_specs=_VMEM,
    )(feats, fc_wt, fc_b.reshape(1, -1))
